# Initial kernel scaffold; baseline (speedup 1.0000x reference)
import jax
import jax.numpy as jnp
from jax.experimental import pallas as pl


def kernel(x, edge_index, edge_attr, e2n_W1, e2n_b1, e2n_W2, e2n_b2, l0_eW1, l0_eb1, l0_eW2, l0_eb2, l0_nW, l0_nb, l1_eW1, l1_eb1, l1_eW2, l1_eb2, l1_nW, l1_nb, out_W1, out_b1, out_W2, out_b2):
    raise NotImplementedError("write your pallas kernel here")


# speedup vs baseline: 4.7281x; 4.7281x over previous
import functools

import jax
import jax.numpy as jnp
from jax import lax
from jax.experimental import pallas as pl
from jax.experimental.pallas import tpu as pltpu
from jax.experimental.pallas import tpu_sc as plsc

N = 100000
E = 1600000
H = 16

NC = 2
NS = 16
NW = NC * NS

SUB = 128
GRP = 4
CHUNK = SUB * GRP
EP = 1605632
EWP = EP // NW
NCHUNK = EWP // CHUNK
IDX_ROWS = EP // SUB
ROWS_PER_W = EWP // SUB

NP = 100352
RPT = NP // NS
ZROWS = 128
ZCOPIES = RPT // ZROWS

_mesh = plsc.VectorSubcoreMesh(core_axis_name="c", subcore_axis_name="s",
                               num_cores=NC, num_subcores=NS)
_sc_params = pltpu.CompilerParams(use_tc_tiling_on_sc=False)


def _sds(shape):
    return jax.ShapeDtypeStruct(shape, jnp.float32)


def _zero_acc(acc, zb, sem, my_rows):
    ds = [pltpu.async_copy(zb, acc.at[pl.ds(my_rows + j * ZROWS, ZROWS)], sem)
          for j in range(ZCOPIES)]
    for d in ds:
        d.wait()


@functools.partial(
    pl.kernel,
    out_type=(_sds((NC, NP, H)), _sds((NC, NP, H))),
    mesh=_mesh,
    scratch_types=[
        pltpu.VMEM((GRP, SUB), jnp.int32),
        pltpu.VMEM((CHUNK, H), jnp.float32),
        pltpu.VMEM((ZROWS, H), jnp.float32),
        pltpu.VMEM_SHARED((NP, H), jnp.float32),
        pltpu.SemaphoreType.DMA,
    ],
    compiler_params=_sc_params,
)
def _sc_scatter0(dst_hbm, a0_hbm, s0_out, cnt_out, idxd, rows, zb, acc, sem):
    c = lax.axis_index("c")
    s = lax.axis_index("s")
    wid = s * NC + c
    my_rows = s * RPT
    my_slice = pl.ds(my_rows, RPT)

    @plsc.parallel_loop(0, ZROWS, unroll=8)
    def _(i):
        zb[i, :] = jnp.zeros((H,), jnp.float32)

    _zero_acc(acc, zb, sem, my_rows)
    plsc.subcore_barrier()

    def chunk_a(i, _):
        rbase = wid * ROWS_PER_W + i * GRP
        ebase = wid * EWP + i * CHUNK
        pltpu.sync_copy(dst_hbm.at[pl.ds(rbase, GRP)], idxd)
        pltpu.async_copy(a0_hbm.at[pl.ds(ebase, CHUNK)], rows, sem).wait()
        for g in range(GRP):
            pltpu.sync_copy(rows.at[pl.ds(g * SUB, SUB)],
                            acc.at[idxd.at[g]], add=True)
        return 0

    lax.fori_loop(0, NCHUNK, chunk_a, 0)
    plsc.subcore_barrier()
    pltpu.sync_copy(acc.at[my_slice], s0_out.at[c, my_slice])
    plsc.subcore_barrier()

    _zero_acc(acc, zb, sem, my_rows)

    @plsc.parallel_loop(0, CHUNK, unroll=8)
    def _(i):
        rows[i, :] = jnp.ones((H,), jnp.float32)

    plsc.subcore_barrier()

    def chunk_b(i, _):
        rbase = wid * ROWS_PER_W + i * GRP
        pltpu.sync_copy(dst_hbm.at[pl.ds(rbase, GRP)], idxd)
        for g in range(GRP):
            pltpu.sync_copy(rows.at[pl.ds(g * SUB, SUB)],
                            acc.at[idxd.at[g]], add=True)
        return 0

    lax.fori_loop(0, NCHUNK, chunk_b, 0)
    plsc.subcore_barrier()
    pltpu.sync_copy(acc.at[my_slice], cnt_out.at[c, my_slice])


@functools.partial(
    pl.kernel,
    out_type=_sds((NC, NP, H)),
    mesh=_mesh,
    scratch_types=[
        pltpu.VMEM((GRP, SUB), jnp.int32),
        pltpu.VMEM((GRP, SUB), jnp.int32),
        pltpu.VMEM((CHUNK, H), jnp.float32),
        pltpu.VMEM((CHUNK, H), jnp.float32),
        pltpu.VMEM((CHUNK, H), jnp.float32),
        pltpu.VMEM((ZROWS, H), jnp.float32),
        pltpu.VMEM_SHARED((NP, H), jnp.float32),
        pltpu.SemaphoreType.DMA,
    ],
    compiler_params=_sc_params,
)
def _sc_layer(dst_hbm, src_hbm, p_hbm, q_hbm, r_hbm, s_out,
              idxd, idxs, pb, qb, rb, zb, acc, sem):
    c = lax.axis_index("c")
    s = lax.axis_index("s")
    wid = s * NC + c
    my_rows = s * RPT
    my_slice = pl.ds(my_rows, RPT)

    @plsc.parallel_loop(0, ZROWS, unroll=8)
    def _(i):
        zb[i, :] = jnp.zeros((H,), jnp.float32)

    _zero_acc(acc, zb, sem, my_rows)
    plsc.subcore_barrier()

    def chunk(i, _):
        rbase = wid * ROWS_PER_W + i * GRP
        ebase = wid * EWP + i * CHUNK
        pltpu.sync_copy(dst_hbm.at[pl.ds(rbase, GRP)], idxd)
        pltpu.sync_copy(src_hbm.at[pl.ds(rbase, GRP)], idxs)
        ds = [pltpu.async_copy(r_hbm.at[pl.ds(ebase, CHUNK)], rb, sem)]
        for g in range(GRP):
            sl = pl.ds(g * SUB, SUB)
            ds.append(pltpu.async_copy(p_hbm.at[idxd.at[g]], pb.at[sl], sem))
            ds.append(pltpu.async_copy(q_hbm.at[idxs.at[g]], qb.at[sl], sem))
        for d in ds:
            d.wait()

        @plsc.parallel_loop(0, CHUNK, unroll=8)
        def _(k):
            pb[k, :] = jnp.maximum(pb[k, :] + qb[k, :] + rb[k, :], 0.0)

        for g in range(GRP):
            pltpu.sync_copy(pb.at[pl.ds(g * SUB, SUB)],
                            acc.at[idxd.at[g]], add=True)
        return 0

    lax.fori_loop(0, NCHUNK, chunk, 0)
    plsc.subcore_barrier()
    pltpu.sync_copy(acc.at[my_slice], s_out.at[c, my_slice])


EB = 512
EBLOCKS = E // EB
EPBLOCKS = EP // EB
NB = 512
NBLOCKS = NP // NB


def _t0_body(ea_ref, w_ref, b_ref, a0_ref, r0_ref, r1_ref):
    y = jnp.dot(ea_ref[...], w_ref[...], preferred_element_type=jnp.float32)
    y = y + b_ref[...]
    a0_ref[...] = jnp.maximum(y[:, :H], 0.0)
    r0_ref[...] = y[:, H:2 * H]
    r1_ref[...] = y[:, 2 * H:]


def _edge_precompute(edge_attr, wcat, bcat):
    return pl.pallas_call(
        _t0_body,
        grid=(EPBLOCKS,),
        in_specs=[
            pl.BlockSpec((EB, H), lambda i: (jnp.minimum(i, EBLOCKS - 1), 0)),
            pl.BlockSpec((H, 3 * H), lambda i: (0, 0)),
            pl.BlockSpec((1, 3 * H), lambda i: (0, 0)),
        ],
        out_specs=[
            pl.BlockSpec((EB, H), lambda i: (i, 0)),
            pl.BlockSpec((EB, H), lambda i: (i, 0)),
            pl.BlockSpec((EB, H), lambda i: (i, 0)),
        ],
        out_shape=[_sds((EP, H)), _sds((EP, H)), _sds((EP, H))],
    )(edge_attr, wcat, bcat)


def _t1_body(s_ref, c_ref, w2_ref, b2_ref, wab_ref, p_ref, q_ref,
             cnt_ref, invd_ref):
    ssum = s_ref[0] + s_ref[1]
    cnt = c_ref[0] + c_ref[1]
    nf = jnp.dot(ssum, w2_ref[...], preferred_element_type=jnp.float32)
    nf = nf + cnt * b2_ref[...]
    pq = jnp.dot(nf, wab_ref[...], preferred_element_type=jnp.float32)
    p_ref[...] = pq[:, :H]
    q_ref[...] = pq[:, H:]
    cnt_ref[...] = cnt
    invd_ref[...] = 1.0 / jnp.maximum(cnt, 1.0)


def _node_init(s0_parts, cnt_parts, w2, b2, wab):
    return pl.pallas_call(
        _t1_body,
        grid=(NBLOCKS,),
        in_specs=[
            pl.BlockSpec((NC, NB, H), lambda i: (0, i, 0)),
            pl.BlockSpec((NC, NB, H), lambda i: (0, i, 0)),
            pl.BlockSpec((H, H), lambda i: (0, 0)),
            pl.BlockSpec((1, H), lambda i: (0, 0)),
            pl.BlockSpec((H, 2 * H), lambda i: (0, 0)),
        ],
        out_specs=[pl.BlockSpec((NB, H), lambda i: (i, 0))] * 4,
        out_shape=[_sds((NP, H))] * 4,
    )(s0_parts, cnt_parts, w2, b2, wab)


def _t2_body(s_ref, cnt_ref, invd_ref, ew2_ref, eb2_ref, nw_ref, nb_ref,
             wab_ref, p_ref, q_ref):
    ssum = s_ref[0] + s_ref[1]
    agg = jnp.dot(ssum, ew2_ref[...], preferred_element_type=jnp.float32)
    agg = (agg + cnt_ref[...] * eb2_ref[...]) * invd_ref[...]
    nf = jnp.dot(agg, nw_ref[...], preferred_element_type=jnp.float32)
    nf = jnp.maximum(nf + nb_ref[...], 0.0)
    pq = jnp.dot(nf, wab_ref[...], preferred_element_type=jnp.float32)
    p_ref[...] = pq[:, :H]
    q_ref[...] = pq[:, H:]


def _node_update(s_parts, cnt, invd, ew2, eb2, nw, nb, wab):
    return pl.pallas_call(
        _t2_body,
        grid=(NBLOCKS,),
        in_specs=[
            pl.BlockSpec((NC, NB, H), lambda i: (0, i, 0)),
            pl.BlockSpec((NB, H), lambda i: (i, 0)),
            pl.BlockSpec((NB, H), lambda i: (i, 0)),
            pl.BlockSpec((H, H), lambda i: (0, 0)),
            pl.BlockSpec((1, H), lambda i: (0, 0)),
            pl.BlockSpec((H, H), lambda i: (0, 0)),
            pl.BlockSpec((1, H), lambda i: (0, 0)),
            pl.BlockSpec((H, 2 * H), lambda i: (0, 0)),
        ],
        out_specs=[pl.BlockSpec((NB, H), lambda i: (i, 0))] * 2,
        out_shape=[_sds((NP, H))] * 2,
    )(s_parts, cnt, invd, ew2, eb2, nw, nb, wab)


def _t3_body(s_ref, cnt_ref, invd_ref, ew2_ref, eb2_ref, nw_ref, nb_ref,
             ow1_ref, ob1_ref, ow2_ref, ob2_ref, out_ref):
    ssum = s_ref[0] + s_ref[1]
    agg = jnp.dot(ssum, ew2_ref[...], preferred_element_type=jnp.float32)
    agg = (agg + cnt_ref[...] * eb2_ref[...]) * invd_ref[...]
    nf = jnp.dot(agg, nw_ref[...], preferred_element_type=jnp.float32)
    nf = jnp.maximum(nf + nb_ref[...], 0.0)
    h = jnp.dot(nf, ow1_ref[...], preferred_element_type=jnp.float32)
    h = jnp.maximum(h + ob1_ref[...], 0.0)
    out_ref[...] = jnp.dot(h, ow2_ref[...],
                           preferred_element_type=jnp.float32) + ob2_ref[...]


def _node_final(s_parts, cnt, invd, ew2, eb2, nw, nb, ow1, ob1, ow2, ob2):
    return pl.pallas_call(
        _t3_body,
        grid=(NBLOCKS,),
        in_specs=[
            pl.BlockSpec((NC, NB, H), lambda i: (0, i, 0)),
            pl.BlockSpec((NB, H), lambda i: (i, 0)),
            pl.BlockSpec((NB, H), lambda i: (i, 0)),
            pl.BlockSpec((H, H), lambda i: (0, 0)),
            pl.BlockSpec((1, H), lambda i: (0, 0)),
            pl.BlockSpec((H, H), lambda i: (0, 0)),
            pl.BlockSpec((1, H), lambda i: (0, 0)),
            pl.BlockSpec((H, H), lambda i: (0, 0)),
            pl.BlockSpec((1, H), lambda i: (0, 0)),
            pl.BlockSpec((H, H), lambda i: (0, 0)),
            pl.BlockSpec((1, H), lambda i: (0, 0)),
        ],
        out_specs=pl.BlockSpec((NB, H), lambda i: (i, 0)),
        out_shape=_sds((NP, H)),
    )(s_parts, cnt, invd, ew2, eb2, nw, nb, ow1, ob1, ow2, ob2)


def kernel(x, edge_index, edge_attr,
           e2n_W1, e2n_b1, e2n_W2, e2n_b2,
           l0_eW1, l0_eb1, l0_eW2, l0_eb2, l0_nW, l0_nb,
           l1_eW1, l1_eb1, l1_eW2, l1_eb2, l1_nW, l1_nb,
           out_W1, out_b1, out_W2, out_b2):
    src = edge_index[0]
    dst = edge_index[1]
    pad = EP - E
    dst_p = jnp.concatenate([dst, jnp.full((pad,), N, jnp.int32)])
    src_p = jnp.concatenate([src, jnp.zeros((pad,), jnp.int32)])
    dst2 = dst_p.reshape(IDX_ROWS, SUB)
    src2 = src_p.reshape(IDX_ROWS, SUB)

    wcat = jnp.concatenate([e2n_W1, l0_eW1[2 * H:], l1_eW1[2 * H:]], axis=1)
    bcat = jnp.concatenate([e2n_b1, l0_eb1, l1_eb1]).reshape(1, 3 * H)
    a0, r0, r1 = _edge_precompute(edge_attr, wcat, bcat)

    s0_parts, cnt_parts = _sc_scatter0(dst2, a0)

    wab0 = jnp.concatenate([l0_eW1[:H], l0_eW1[H:2 * H]], axis=1)
    p0, q0, cnt, invd = _node_init(
        s0_parts, cnt_parts, e2n_W2, e2n_b2.reshape(1, H), wab0)

    s_parts = _sc_layer(dst2, src2, p0, q0, r0)
    wab1 = jnp.concatenate([l1_eW1[:H], l1_eW1[H:2 * H]], axis=1)
    p1, q1 = _node_update(s_parts, cnt, invd, l0_eW2, l0_eb2.reshape(1, H),
                          l0_nW, l0_nb.reshape(1, H), wab1)

    s_parts = _sc_layer(dst2, src2, p1, q1, r1)
    pred = _node_final(s_parts, cnt, invd, l1_eW2, l1_eb2.reshape(1, H),
                       l1_nW, l1_nb.reshape(1, H),
                       out_W1, out_b1.reshape(1, H),
                       out_W2, out_b2.reshape(1, H))
    return pred[:N]


# speedup vs baseline: 12.8231x; 2.7121x over previous
import functools

import jax
import jax.numpy as jnp
from jax import lax
from jax.experimental import pallas as pl
from jax.experimental.pallas import tpu as pltpu
from jax.experimental.pallas import tpu_sc as plsc

N = 100000
E = 1600000
H = 16

NC = 2
NS = 16
NW = NC * NS

SUB = 128
GRP = 4
CHUNK = SUB * GRP
NCH = E // CHUNK
CH_HI = 98
CH_LO = 97
W_HI = NCH - NW * CH_LO
IDX_ROWS = E // SUB

RPT = N // NS
ZROWS = 125
ZCOPIES = RPT // ZROWS

_mesh = plsc.VectorSubcoreMesh(core_axis_name="c", subcore_axis_name="s",
                               num_cores=NC, num_subcores=NS)
_sc_params = pltpu.CompilerParams(use_tc_tiling_on_sc=False)


def _sds(shape):
    return jax.ShapeDtypeStruct(shape, jnp.float32)


def _worker_chunks(wid):
    nb = jnp.where(wid < W_HI, CH_HI, CH_LO)
    cbase = jnp.where(wid < W_HI, wid * CH_HI,
                      W_HI * CH_HI + (wid - W_HI) * CH_LO)
    return nb, cbase


def _zero_acc(acc, zb, sem, my_rows):
    ds = [pltpu.async_copy(zb, acc.at[pl.ds(my_rows + j * ZROWS, ZROWS)], sem)
          for j in range(ZCOPIES)]
    for d in ds:
        d.wait()


@functools.partial(
    pl.kernel,
    out_type=(_sds((NC, N, H)), _sds((NC, N, H))),
    mesh=_mesh,
    scratch_types=[
        pltpu.VMEM((GRP, SUB), jnp.int32),
        pltpu.VMEM((CHUNK, H), jnp.float32),
        pltpu.VMEM((ZROWS, H), jnp.float32),
        pltpu.VMEM_SHARED((N, H), jnp.float32),
        pltpu.SemaphoreType.DMA,
    ],
    compiler_params=_sc_params,
)
def _sc_scatter0(ei_hbm, a0_hbm, s0_out, cnt_out, idxd, rows, zb, acc, sem):
    c = lax.axis_index("c")
    s = lax.axis_index("s")
    wid = s * NC + c
    nb, cbase = _worker_chunks(wid)
    my_rows = s * RPT
    my_slice = pl.ds(my_rows, RPT)

    @plsc.parallel_loop(0, ZROWS, unroll=8)
    def _(i):
        zb[i, :] = jnp.zeros((H,), jnp.float32)

    _zero_acc(acc, zb, sem, my_rows)
    plsc.subcore_barrier()

    def chunk_a(i, _):
        g = cbase + i
        pltpu.sync_copy(ei_hbm.at[1, pl.ds(g * GRP, GRP)], idxd)
        pltpu.async_copy(a0_hbm.at[pl.ds(g * CHUNK, CHUNK)], rows, sem).wait()
        for j in range(GRP):
            pltpu.sync_copy(rows.at[pl.ds(j * SUB, SUB)],
                            acc.at[idxd.at[j]], add=True)
        return 0

    lax.fori_loop(0, nb, chunk_a, 0)
    plsc.subcore_barrier()
    pltpu.sync_copy(acc.at[my_slice], s0_out.at[c, my_slice])
    plsc.subcore_barrier()

    _zero_acc(acc, zb, sem, my_rows)

    @plsc.parallel_loop(0, CHUNK, unroll=8)
    def _(i):
        rows[i, :] = jnp.ones((H,), jnp.float32)

    plsc.subcore_barrier()

    def chunk_b(i, _):
        g = cbase + i
        pltpu.sync_copy(ei_hbm.at[1, pl.ds(g * GRP, GRP)], idxd)
        for j in range(GRP):
            pltpu.sync_copy(rows.at[pl.ds(j * SUB, SUB)],
                            acc.at[idxd.at[j]], add=True)
        return 0

    lax.fori_loop(0, nb, chunk_b, 0)
    plsc.subcore_barrier()
    pltpu.sync_copy(acc.at[my_slice], cnt_out.at[c, my_slice])


@functools.partial(
    pl.kernel,
    out_type=_sds((NC, N, H)),
    mesh=_mesh,
    scratch_types=[
        pltpu.VMEM((GRP, SUB), jnp.int32),
        pltpu.VMEM((GRP, SUB), jnp.int32),
        pltpu.VMEM((CHUNK, H), jnp.float32),
        pltpu.VMEM((CHUNK, H), jnp.float32),
        pltpu.VMEM((CHUNK, H), jnp.float32),
        pltpu.VMEM((ZROWS, H), jnp.float32),
        pltpu.VMEM_SHARED((N, H), jnp.float32),
        pltpu.SemaphoreType.DMA,
    ],
    compiler_params=_sc_params,
)
def _sc_layer(ei_hbm, p_hbm, q_hbm, r_hbm, s_out,
              idxd, idxs, pb, qb, rb, zb, acc, sem):
    c = lax.axis_index("c")
    s = lax.axis_index("s")
    wid = s * NC + c
    nb, cbase = _worker_chunks(wid)
    my_rows = s * RPT
    my_slice = pl.ds(my_rows, RPT)

    @plsc.parallel_loop(0, ZROWS, unroll=8)
    def _(i):
        zb[i, :] = jnp.zeros((H,), jnp.float32)

    _zero_acc(acc, zb, sem, my_rows)
    plsc.subcore_barrier()

    def chunk(i, _):
        g = cbase + i
        pltpu.sync_copy(ei_hbm.at[1, pl.ds(g * GRP, GRP)], idxd)
        pltpu.sync_copy(ei_hbm.at[0, pl.ds(g * GRP, GRP)], idxs)
        ds = [pltpu.async_copy(r_hbm.at[pl.ds(g * CHUNK, CHUNK)], rb, sem)]
        for j in range(GRP):
            sl = pl.ds(j * SUB, SUB)
            ds.append(pltpu.async_copy(p_hbm.at[idxd.at[j]], pb.at[sl], sem))
            ds.append(pltpu.async_copy(q_hbm.at[idxs.at[j]], qb.at[sl], sem))
        for d in ds:
            d.wait()

        @plsc.parallel_loop(0, CHUNK, unroll=8)
        def _(k):
            pb[k, :] = jnp.maximum(pb[k, :] + qb[k, :] + rb[k, :], 0.0)

        for j in range(GRP):
            pltpu.sync_copy(pb.at[pl.ds(j * SUB, SUB)],
                            acc.at[idxd.at[j]], add=True)
        return 0

    lax.fori_loop(0, nb, chunk, 0)
    plsc.subcore_barrier()
    pltpu.sync_copy(acc.at[my_slice], s_out.at[c, my_slice])


ER = E // 8
EB = 1000
EBLOCKS = ER // EB
NR = N // 8
NB = 512
NBLOCKS = -(-NR // NB)


def _t0_body(ea_ref, w_ref, b_ref, a0_ref, r0_ref, r1_ref):
    y = jnp.dot(ea_ref[...], w_ref[...], preferred_element_type=jnp.float32)
    y = y + b_ref[...]
    a0_ref[...] = jnp.maximum(y[:, :128], 0.0)
    r0_ref[...] = y[:, 128:256]
    r1_ref[...] = y[:, 256:]


def _edge_precompute(ea_r, wbd, btile):
    return pl.pallas_call(
        _t0_body,
        grid=(EBLOCKS,),
        in_specs=[
            pl.BlockSpec((EB, 128), lambda i: (i, 0)),
            pl.BlockSpec((128, 384), lambda i: (0, 0)),
            pl.BlockSpec((1, 384), lambda i: (0, 0)),
        ],
        out_specs=[pl.BlockSpec((EB, 128), lambda i: (i, 0))] * 3,
        out_shape=[_sds((ER, 128))] * 3,
    )(ea_r, wbd, btile)


def _t1_body(s_ref, c_ref, w2_ref, b2_ref, wab_ref, p_ref, q_ref,
             cnt_ref, invd_ref):
    ssum = s_ref[0] + s_ref[1]
    cnt = c_ref[0] + c_ref[1]
    nf = jnp.dot(ssum, w2_ref[...], preferred_element_type=jnp.float32)
    nf = nf + cnt * b2_ref[...]
    pq = jnp.dot(nf, wab_ref[...], preferred_element_type=jnp.float32)
    p_ref[...] = pq[:, :128]
    q_ref[...] = pq[:, 128:]
    cnt_ref[...] = cnt
    invd_ref[...] = 1.0 / jnp.maximum(cnt, 1.0)


def _node_init(s0_parts, cnt_parts, w2bd, b2tile, wabbd):
    return pl.pallas_call(
        _t1_body,
        grid=(NBLOCKS,),
        in_specs=[
            pl.BlockSpec((NC, NB, 128), lambda i: (0, i, 0)),
            pl.BlockSpec((NC, NB, 128), lambda i: (0, i, 0)),
            pl.BlockSpec((128, 128), lambda i: (0, 0)),
            pl.BlockSpec((1, 128), lambda i: (0, 0)),
            pl.BlockSpec((128, 256), lambda i: (0, 0)),
        ],
        out_specs=[pl.BlockSpec((NB, 128), lambda i: (i, 0))] * 4,
        out_shape=[_sds((NR, 128))] * 4,
    )(s0_parts, cnt_parts, w2bd, b2tile, wabbd)


def _t2_body(s_ref, cnt_ref, invd_ref, ew2_ref, eb2_ref, nw_ref, nb_ref,
             wab_ref, p_ref, q_ref):
    ssum = s_ref[0] + s_ref[1]
    agg = jnp.dot(ssum, ew2_ref[...], preferred_element_type=jnp.float32)
    agg = (agg + cnt_ref[...] * eb2_ref[...]) * invd_ref[...]
    nf = jnp.dot(agg, nw_ref[...], preferred_element_type=jnp.float32)
    nf = jnp.maximum(nf + nb_ref[...], 0.0)
    pq = jnp.dot(nf, wab_ref[...], preferred_element_type=jnp.float32)
    p_ref[...] = pq[:, :128]
    q_ref[...] = pq[:, 128:]


def _node_update(s_parts, cnt, invd, ew2bd, eb2t, nwbd, nbt, wabbd):
    return pl.pallas_call(
        _t2_body,
        grid=(NBLOCKS,),
        in_specs=[
            pl.BlockSpec((NC, NB, 128), lambda i: (0, i, 0)),
            pl.BlockSpec((NB, 128), lambda i: (i, 0)),
            pl.BlockSpec((NB, 128), lambda i: (i, 0)),
            pl.BlockSpec((128, 128), lambda i: (0, 0)),
            pl.BlockSpec((1, 128), lambda i: (0, 0)),
            pl.BlockSpec((128, 128), lambda i: (0, 0)),
            pl.BlockSpec((1, 128), lambda i: (0, 0)),
            pl.BlockSpec((128, 256), lambda i: (0, 0)),
        ],
        out_specs=[pl.BlockSpec((NB, 128), lambda i: (i, 0))] * 2,
        out_shape=[_sds((NR, 128))] * 2,
    )(s_parts, cnt, invd, ew2bd, eb2t, nwbd, nbt, wabbd)


def _t3_body(s_ref, cnt_ref, invd_ref, ew2_ref, eb2_ref, nw_ref, nb_ref,
             ow1_ref, ob1_ref, ow2_ref, ob2_ref, out_ref):
    ssum = s_ref[0] + s_ref[1]
    agg = jnp.dot(ssum, ew2_ref[...], preferred_element_type=jnp.float32)
    agg = (agg + cnt_ref[...] * eb2_ref[...]) * invd_ref[...]
    nf = jnp.dot(agg, nw_ref[...], preferred_element_type=jnp.float32)
    nf = jnp.maximum(nf + nb_ref[...], 0.0)
    h = jnp.dot(nf, ow1_ref[...], preferred_element_type=jnp.float32)
    h = jnp.maximum(h + ob1_ref[...], 0.0)
    out_ref[...] = jnp.dot(h, ow2_ref[...],
                           preferred_element_type=jnp.float32) + ob2_ref[...]


def _node_final(s_parts, cnt, invd, ew2bd, eb2t, nwbd, nbt,
                ow1bd, ob1t, ow2bd, ob2t):
    return pl.pallas_call(
        _t3_body,
        grid=(NBLOCKS,),
        in_specs=[
            pl.BlockSpec((NC, NB, 128), lambda i: (0, i, 0)),
            pl.BlockSpec((NB, 128), lambda i: (i, 0)),
            pl.BlockSpec((NB, 128), lambda i: (i, 0)),
            pl.BlockSpec((128, 128), lambda i: (0, 0)),
            pl.BlockSpec((1, 128), lambda i: (0, 0)),
            pl.BlockSpec((128, 128), lambda i: (0, 0)),
            pl.BlockSpec((1, 128), lambda i: (0, 0)),
            pl.BlockSpec((128, 128), lambda i: (0, 0)),
            pl.BlockSpec((1, 128), lambda i: (0, 0)),
            pl.BlockSpec((128, 128), lambda i: (0, 0)),
            pl.BlockSpec((1, 128), lambda i: (0, 0)),
        ],
        out_specs=pl.BlockSpec((NB, 128), lambda i: (i, 0)),
        out_shape=_sds((NR, 128)),
    )(s_parts, cnt, invd, ew2bd, eb2t, nwbd, nbt, ow1bd, ob1t, ow2bd, ob2t)


def _bd(w):
    return jnp.kron(jnp.eye(8, dtype=jnp.float32), w)


def _tile8(b):
    return jnp.tile(b, 8).reshape(1, 128)


def kernel(x, edge_index, edge_attr,
           e2n_W1, e2n_b1, e2n_W2, e2n_b2,
           l0_eW1, l0_eb1, l0_eW2, l0_eb2, l0_nW, l0_nb,
           l1_eW1, l1_eb1, l1_eW2, l1_eb2, l1_nW, l1_nb,
           out_W1, out_b1, out_W2, out_b2):
    ei3 = edge_index.reshape(2, IDX_ROWS, SUB)
    ea_r = edge_attr.reshape(ER, 128)

    wbd = jnp.concatenate(
        [_bd(e2n_W1), _bd(l0_eW1[2 * H:]), _bd(l1_eW1[2 * H:])], axis=1)
    btile = jnp.concatenate(
        [_tile8(e2n_b1), _tile8(l0_eb1), _tile8(l1_eb1)], axis=1)
    a0, r0, r1 = _edge_precompute(ea_r, wbd, btile)

    s0_parts, cnt_parts = _sc_scatter0(ei3, a0.reshape(E, H))

    wab0 = jnp.concatenate([_bd(l0_eW1[:H]), _bd(l0_eW1[H:2 * H])], axis=1)
    p0, q0, cnt, invd = _node_init(
        s0_parts.reshape(NC, NR, 128), cnt_parts.reshape(NC, NR, 128),
        _bd(e2n_W2), _tile8(e2n_b2), wab0)

    s_parts = _sc_layer(ei3, p0.reshape(N, H), q0.reshape(N, H),
                        r0.reshape(E, H))
    wab1 = jnp.concatenate([_bd(l1_eW1[:H]), _bd(l1_eW1[H:2 * H])], axis=1)
    p1, q1 = _node_update(s_parts.reshape(NC, NR, 128), cnt, invd,
                          _bd(l0_eW2), _tile8(l0_eb2),
                          _bd(l0_nW), _tile8(l0_nb), wab1)

    s_parts = _sc_layer(ei3, p1.reshape(N, H), q1.reshape(N, H),
                        r1.reshape(E, H))
    pred = _node_final(s_parts.reshape(NC, NR, 128), cnt, invd,
                       _bd(l1_eW2), _tile8(l1_eb2),
                       _bd(l1_nW), _tile8(l1_nb),
                       _bd(out_W1), _tile8(out_b1),
                       _bd(out_W2), _tile8(out_b2))
    return pred.reshape(N, H)
